# baseline (speedup 1.0000x reference)
import jax
import jax.numpy as jnp
from jax import lax
from jax.experimental import pallas as pl

F32 = jnp.float32

QROWS = 232
PHW1 = 64
NF1 = QROWS * PHW1
TILE_A = NF1 // 2
RP = 19
PHW2 = 64


def _matmul(a, w):
    return lax.dot_general(a.astype(jnp.bfloat16), w.astype(jnp.bfloat16),
                           (((a.ndim - 1,), (0,)), ((), ())),
                           preferred_element_type=F32)


def _silu(x):
    return x * jax.nn.sigmoid(x)


def _stageA_body(x_ref, w_ref, b_ref, m_ref, o_ref):
    acc = _matmul(x_ref[0, 0], w_ref[...])
    acc = _silu(acc + b_ref[0][None, :])
    o_ref[0, 0] = acc * m_ref[0]


def _stageA(xcat4, w1f, b1, mask):
    return pl.pallas_call(
        _stageA_body,
        grid=(4, 4, NF1 // TILE_A),
        in_specs=[
            pl.BlockSpec((1, 1, TILE_A, 54), lambda n, k, t: (n, k, t, 0)),
            pl.BlockSpec((54, 128), lambda n, k, t: (0, 0)),
            pl.BlockSpec((1, 128), lambda n, k, t: (0, 0)),
            pl.BlockSpec((1, TILE_A, 1), lambda n, k, t: (k, t, 0)),
        ],
        out_specs=pl.BlockSpec((1, 1, TILE_A, 128), lambda n, k, t: (n, k, t, 0)),
        out_shape=jax.ShapeDtypeStruct((4, 4, NF1, 128), F32),
    )(xcat4, w1f, b1, mask)


def _stageB_body(m_ref, h_ref, w_ref, b_ref, o_ref):
    mr = m_ref[0].reshape(4, RP, 2, PHW1, 128)
    ev = mr[:, :, 0]
    od = mr[:, :, 1]
    halo = h_ref[0]
    slab = [ev, od, jnp.concatenate([ev[:, 1:RP], halo], axis=1)]
    bias = b_ref[0][None, :]
    t = pl.program_id(1)
    i2 = lax.broadcasted_iota(jnp.int32, (RP * PHW1, 1), 0)
    prow = t * RP + i2 // PHW1
    jcol = i2 % PHW1
    rowok = (prow >= 1) & (prow < 113)
    zero = jnp.zeros((RP, 1, 128), F32)
    for par in range(2):
        acc = jnp.zeros((RP * PHW1, 128), F32)
        for dy in range(3):
            for dx in range(3):
                if par == 0:
                    if dx < 2:
                        a = jnp.concatenate([zero, slab[dy][2 + dx, :, 0:PHW1 - 1]], axis=1)
                    else:
                        a = slab[dy][0]
                else:
                    a = slab[dy][dx]
                acc = acc + _matmul(a.reshape(RP * PHW1, 128), w_ref[dy * 3 + dx])
        acc = _silu(acc + bias)
        if par == 0:
            ok = rowok & (jcol >= 1) & (jcol < 57)
        else:
            ok = rowok & (jcol < 56)
        acc = acc * ok.astype(F32)
        o_ref[0, par] = acc.reshape(RP, PHW1, 128)


def _stageB(z1sq, w2f, b2):
    return pl.pallas_call(
        _stageB_body,
        grid=(4, 114 // RP),
        in_specs=[
            pl.BlockSpec((1, 4, 2 * RP, PHW1, 128), lambda n, t: (n, 0, t, 0, 0)),
            pl.BlockSpec((1, 4, 1, PHW1, 128), lambda n, t: (n, 0, (t + 1) * 2 * RP, 0, 0)),
            pl.BlockSpec((9, 128, 128), lambda n, t: (0, 0, 0)),
            pl.BlockSpec((1, 128), lambda n, t: (0, 0)),
        ],
        out_specs=pl.BlockSpec((1, 2, RP, PHW1, 128), lambda n, t: (n, 0, t, 0, 0)),
        out_shape=jax.ShapeDtypeStruct((4, 2, 114, PHW1, 128), F32),
    )(z1sq, z1sq, w2f, b2)


def _stageC_body(z_ref, w_ref, b_ref, cbt_ref, tok_ref, loss_ref):
    e2 = z_ref[0, 0].reshape(57, 2, PHW2, 128)
    o2 = z_ref[0, 1].reshape(57, 2, PHW2, 128)
    erow = [e2[:, 0], o2[:, 0]]
    orow = [e2[:, 1], o2[:, 1]]
    acc = jnp.zeros((56 * 56, 64), F32)
    for dy in range(3):
        for dx in range(3):
            if dy == 0:
                rs = [erow[0][0:56], erow[1][0:56]]
            elif dy == 1:
                rs = [orow[0][0:56], orow[1][0:56]]
            else:
                rs = [erow[0][1:57], erow[1][1:57]]
            if dx == 0:
                a = rs[0][:, 0:56]
            elif dx == 1:
                a = rs[1][:, 0:56]
            else:
                a = rs[0][:, 1:57]
            acc = acc + _matmul(a.reshape(56 * 56, 128), w_ref[dy * 3 + dx])
    z3 = acc + b_ref[0][None, :]
    cbt = cbt_ref[...]
    csq = jnp.sum(cbt * cbt, axis=0, keepdims=True)
    zsq = jnp.sum(z3 * z3, axis=1, keepdims=True)
    dots = _matmul(z3, cbt)
    dists = zsq + csq - 2.0 * dots
    tok_ref[0, 0] = jnp.argmin(dists, axis=1).astype(jnp.int32)
    loss_ref[0] = jnp.sum(jnp.min(dists, axis=1))[None, None]


def _stageC(z2s, w3f, b3, cbt):
    return pl.pallas_call(
        _stageC_body,
        grid=(4,),
        in_specs=[
            pl.BlockSpec((1, 2, 114, PHW2, 128), lambda n: (n, 0, 0, 0, 0)),
            pl.BlockSpec((9, 128, 64), lambda n: (0, 0, 0)),
            pl.BlockSpec((1, 64), lambda n: (0, 0)),
            pl.BlockSpec((64, 512), lambda n: (0, 0)),
        ],
        out_specs=[
            pl.BlockSpec((1, 1, 3136), lambda n: (n, 0, 0)),
            pl.BlockSpec((1, 1, 1), lambda n: (n, 0, 0)),
        ],
        out_shape=[
            jax.ShapeDtypeStruct((4, 1, 3136), jnp.int32),
            jax.ShapeDtypeStruct((4, 1, 1), F32),
        ],
    )(z2s, w3f, b3, cbt)


def kernel(force_field, w1, b1, w2, b2, w3, b3, codebook):
    x = jnp.transpose(force_field, (0, 2, 3, 1))
    xp = jnp.pad(x, ((0, 0), (4, 10), (2, 34), (0, 0)))
    phases = []
    for k in range(4):
        shifts = [lax.slice(xp, (0, dy, k + dx, 0), (4, dy + QROWS, k + dx + 253, 6),
                            (1, 1, 4, 1))
                  for dy in range(3) for dx in range(3)]
        phases.append(jnp.concatenate(shifts, axis=3))
    xcat4 = jnp.stack(phases, axis=1).reshape(4, 4, NF1, 54)

    q = jnp.arange(NF1) // PHW1
    j4 = jnp.arange(NF1) % PHW1
    masks = []
    for k in range(4):
        c = 4 * j4 + k
        masks.append(((q >= 3) & (q < 227) & (c >= 1) & (c < 225)).astype(F32))
    mask = jnp.stack(masks, axis=0)[:, :, None]

    w1f = jnp.transpose(w1, (2, 3, 1, 0)).reshape(54, 128)
    w2f = jnp.transpose(w2, (2, 3, 1, 0)).reshape(9, 128, 128)
    w3f = jnp.transpose(w3, (2, 3, 1, 0)).reshape(9, 128, 64)
    b1r, b2r, b3r = b1[None, :], b2[None, :], b3[None, :]
    cbt = jnp.transpose(codebook, (1, 0))

    z1sq = _stageA(xcat4, w1f, b1r, mask)
    z1sq = z1sq.reshape(4, 4, QROWS, PHW1, 128)
    z2s = _stageB(z1sq, w2f, b2r)
    tok, loss_part = _stageC(z2s, w3f, b3r, cbt)

    tokens = tok.reshape(4, 56, 56)
    loss = (1.0 + 0.25) * jnp.sum(loss_part) / (4 * 64 * 56 * 56)
    return tokens, loss


# speedup vs baseline: 2.6027x; 2.6027x over previous
import jax
import jax.numpy as jnp
from jax import lax
from jax.experimental import pallas as pl

F32 = jnp.float32

QROWS = 232
PHW1 = 64
NF1 = QROWS * PHW1
TILE_A = NF1 // 2
RP = 19
PHW2 = 64


def _matmul(a, w):
    return lax.dot_general(a.astype(jnp.bfloat16), w.astype(jnp.bfloat16),
                           (((a.ndim - 1,), (0,)), ((), ())),
                           preferred_element_type=F32)


def _silu(x):
    return x * jax.nn.sigmoid(x)


def _stageA_body(x_ref, w_ref, b_ref, m_ref, o_ref):
    acc = _matmul(x_ref[0, 0], w_ref[...])
    acc = _silu(acc + b_ref[0][None, :])
    o_ref[0, 0] = acc * m_ref[0]


def _stageA(xcat4, w1f, b1, mask):
    return pl.pallas_call(
        _stageA_body,
        grid=(4, 4, NF1 // TILE_A),
        in_specs=[
            pl.BlockSpec((1, 1, TILE_A, 54), lambda n, k, t: (n, k, t, 0)),
            pl.BlockSpec((54, 128), lambda n, k, t: (0, 0)),
            pl.BlockSpec((1, 128), lambda n, k, t: (0, 0)),
            pl.BlockSpec((1, TILE_A, 1), lambda n, k, t: (k, t, 0)),
        ],
        out_specs=pl.BlockSpec((1, 1, TILE_A, 128), lambda n, k, t: (n, k, t, 0)),
        out_shape=jax.ShapeDtypeStruct((4, 4, NF1, 128), F32),
    )(xcat4, w1f, b1, mask)


def _stageB_body(m_ref, h_ref, w_ref, b_ref, o_ref):
    mr = m_ref[0].reshape(4, RP, 2, PHW1, 128)
    ev = mr[:, :, 0]
    od = mr[:, :, 1]
    halo = h_ref[0]
    slab = [ev, od, jnp.concatenate([ev[:, 1:RP], halo], axis=1)]
    bias = b_ref[0][None, :]
    t = pl.program_id(1)
    i2 = lax.broadcasted_iota(jnp.int32, (RP * PHW1, 1), 0)
    prow = t * RP + i2 // PHW1
    jcol = i2 % PHW1
    rowok = (prow >= 1) & (prow < 113)
    zero = jnp.zeros((RP, 1, 128), F32)
    for par in range(2):
        acc = jnp.zeros((RP * PHW1, 128), F32)
        for dy in range(3):
            for dx in range(3):
                if par == 0:
                    if dx < 2:
                        a = jnp.concatenate([zero, slab[dy][2 + dx, :, 0:PHW1 - 1]], axis=1)
                    else:
                        a = slab[dy][0]
                else:
                    a = slab[dy][dx]
                acc = acc + _matmul(a.reshape(RP * PHW1, 128), w_ref[dy * 3 + dx])
        acc = _silu(acc + bias)
        if par == 0:
            ok = rowok & (jcol >= 1) & (jcol < 57)
        else:
            ok = rowok & (jcol < 56)
        acc = acc * ok.astype(F32)
        o_ref[0, par] = acc.reshape(RP, PHW1, 128)


def _stageB(z1sq, w2f, b2):
    return pl.pallas_call(
        _stageB_body,
        grid=(4, 114 // RP),
        in_specs=[
            pl.BlockSpec((1, 4, 2 * RP, PHW1, 128), lambda n, t: (n, 0, t, 0, 0)),
            pl.BlockSpec((1, 4, 1, PHW1, 128), lambda n, t: (n, 0, (t + 1) * 2 * RP, 0, 0)),
            pl.BlockSpec((9, 128, 128), lambda n, t: (0, 0, 0)),
            pl.BlockSpec((1, 128), lambda n, t: (0, 0)),
        ],
        out_specs=pl.BlockSpec((1, 2, RP, PHW1, 128), lambda n, t: (n, 0, t, 0, 0)),
        out_shape=jax.ShapeDtypeStruct((4, 2, 114, PHW1, 128), F32),
    )(z1sq, z1sq, w2f, b2)


def _stageC_body(z_ref, w_ref, b_ref, cbt_ref, tok_ref, loss_ref):
    e2 = z_ref[0, 0].reshape(57, 2, PHW2, 128)
    o2 = z_ref[0, 1].reshape(57, 2, PHW2, 128)
    erow = [e2[:, 0], o2[:, 0]]
    orow = [e2[:, 1], o2[:, 1]]
    acc = jnp.zeros((56 * 56, 64), F32)
    for dy in range(3):
        for dx in range(3):
            if dy == 0:
                rs = [erow[0][0:56], erow[1][0:56]]
            elif dy == 1:
                rs = [orow[0][0:56], orow[1][0:56]]
            else:
                rs = [erow[0][1:57], erow[1][1:57]]
            if dx == 0:
                a = rs[0][:, 0:56]
            elif dx == 1:
                a = rs[1][:, 0:56]
            else:
                a = rs[0][:, 1:57]
            acc = acc + _matmul(a.reshape(56 * 56, 128), w_ref[dy * 3 + dx])
    z3 = acc + b_ref[0][None, :]
    cbt = cbt_ref[...]
    csq = jnp.sum(cbt * cbt, axis=0, keepdims=True)
    zsq = jnp.sum(z3 * z3, axis=1, keepdims=True)
    dots = _matmul(z3, cbt)
    dists = zsq + csq - 2.0 * dots
    tok_ref[0, 0] = jnp.argmin(dists, axis=1).astype(jnp.int32)
    loss_ref[0] = jnp.sum(jnp.min(dists, axis=1))[None, None]


def _stageC(z2s, w3f, b3, cbt):
    return pl.pallas_call(
        _stageC_body,
        grid=(4,),
        in_specs=[
            pl.BlockSpec((1, 2, 114, PHW2, 128), lambda n: (n, 0, 0, 0, 0)),
            pl.BlockSpec((9, 128, 64), lambda n: (0, 0, 0)),
            pl.BlockSpec((1, 64), lambda n: (0, 0)),
            pl.BlockSpec((64, 512), lambda n: (0, 0)),
        ],
        out_specs=[
            pl.BlockSpec((1, 1, 3136), lambda n: (n, 0, 0)),
            pl.BlockSpec((1, 1, 1), lambda n: (n, 0, 0)),
        ],
        out_shape=[
            jax.ShapeDtypeStruct((4, 1, 3136), jnp.int32),
            jax.ShapeDtypeStruct((4, 1, 1), F32),
        ],
    )(z2s, w3f, b3, cbt)


def kernel(force_field, w1, b1, w2, b2, w3, b3, codebook):
    x = jnp.transpose(force_field, (0, 2, 3, 1))
    xp = jnp.pad(x, ((0, 0), (4, 10), (2, 34), (0, 0)))
    xph = xp.reshape(4, 238, 65, 4, 6).transpose(0, 3, 1, 2, 4)
    phases = []
    for k in range(4):
        shifts = []
        for dy in range(3):
            for dx in range(3):
                pp, joff = (k + dx) % 4, (k + dx) // 4
                shifts.append(lax.slice(
                    xph, (0, pp, dy, joff, 0),
                    (4, pp + 1, dy + QROWS, joff + PHW1, 6))[:, 0])
        phases.append(jnp.concatenate(shifts, axis=3))
    xcat4 = jnp.stack(phases, axis=1).reshape(4, 4, NF1, 54)

    q = jnp.arange(NF1) // PHW1
    j4 = jnp.arange(NF1) % PHW1
    masks = []
    for k in range(4):
        c = 4 * j4 + k
        masks.append(((q >= 3) & (q < 227) & (c >= 1) & (c < 225)).astype(F32))
    mask = jnp.stack(masks, axis=0)[:, :, None]

    w1f = jnp.transpose(w1, (2, 3, 1, 0)).reshape(54, 128)
    w2f = jnp.transpose(w2, (2, 3, 1, 0)).reshape(9, 128, 128)
    w3f = jnp.transpose(w3, (2, 3, 1, 0)).reshape(9, 128, 64)
    b1r, b2r, b3r = b1[None, :], b2[None, :], b3[None, :]
    cbt = jnp.transpose(codebook, (1, 0))

    z1sq = _stageA(xcat4, w1f, b1r, mask)
    z1sq = z1sq.reshape(4, 4, QROWS, PHW1, 128)
    z2s = _stageB(z1sq, w2f, b2r)
    tok, loss_part = _stageC(z2s, w3f, b3r, cbt)

    tokens = tok.reshape(4, 56, 56)
    loss = (1.0 + 0.25) * jnp.sum(loss_part) / (4 * 64 * 56 * 56)
    return tokens, loss


# speedup vs baseline: 3.4552x; 1.3275x over previous
import jax
import jax.numpy as jnp
from jax import lax
from jax.experimental import pallas as pl

F32 = jnp.float32

QROWS = 232
PHW1 = 64
NF1 = QROWS * PHW1
TILE_A = NF1 // 2
RP = 19
PHW2 = 64


def _matmul(a, w):
    return lax.dot_general(a.astype(jnp.bfloat16), w.astype(jnp.bfloat16),
                           (((a.ndim - 1,), (0,)), ((), ())),
                           preferred_element_type=F32)


def _silu(x):
    return x * jax.nn.sigmoid(x)


def _stageA_body(x_ref, w_ref, b_ref, m_ref, o_ref):
    acc = _matmul(x_ref[0, 0], w_ref[...])
    acc = _silu(acc + b_ref[0][None, :])
    o_ref[0, 0] = (acc * m_ref[0]).astype(jnp.bfloat16)


def _stageA(xcat4, w1f, b1, mask):
    return pl.pallas_call(
        _stageA_body,
        grid=(4, 4, NF1 // TILE_A),
        in_specs=[
            pl.BlockSpec((1, 1, TILE_A, 54), lambda n, k, t: (n, k, t, 0)),
            pl.BlockSpec((54, 128), lambda n, k, t: (0, 0)),
            pl.BlockSpec((1, 128), lambda n, k, t: (0, 0)),
            pl.BlockSpec((1, TILE_A, 1), lambda n, k, t: (k, t, 0)),
        ],
        out_specs=pl.BlockSpec((1, 1, TILE_A, 128), lambda n, k, t: (n, k, t, 0)),
        out_shape=jax.ShapeDtypeStruct((4, 4, NF1, 128), jnp.bfloat16),
    )(xcat4, w1f, b1, mask)


def _stageB_body(m_ref, h_ref, w_ref, b_ref, o_ref):
    mr = m_ref[0].reshape(4, RP, 2, PHW1, 128)
    ev = mr[:, :, 0]
    od = mr[:, :, 1]
    halo = h_ref[0]
    slab = [ev, od, jnp.concatenate([ev[:, 1:RP], halo], axis=1)]
    bias = b_ref[0][None, :]
    t = pl.program_id(1)
    i2 = lax.broadcasted_iota(jnp.int32, (RP * PHW1, 1), 0)
    prow = t * RP + i2 // PHW1
    jcol = i2 % PHW1
    rowok = (prow >= 1) & (prow < 113)
    zero = jnp.zeros((RP, 1, 128), jnp.bfloat16)
    for par in range(2):
        acc = jnp.zeros((RP * PHW1, 128), F32)
        for dy in range(3):
            for dx in range(3):
                if par == 0:
                    if dx < 2:
                        a = jnp.concatenate([zero, slab[dy][2 + dx, :, 0:PHW1 - 1]], axis=1)
                    else:
                        a = slab[dy][0]
                else:
                    a = slab[dy][dx]
                acc = acc + _matmul(a.reshape(RP * PHW1, 128), w_ref[dy * 3 + dx])
        acc = _silu(acc + bias)
        if par == 0:
            ok = rowok & (jcol >= 1) & (jcol < 57)
        else:
            ok = rowok & (jcol < 56)
        acc = acc * ok.astype(F32)
        o_ref[0, par] = acc.reshape(RP, PHW1, 128).astype(jnp.bfloat16)


def _stageB(z1sq, w2f, b2):
    return pl.pallas_call(
        _stageB_body,
        grid=(4, 114 // RP),
        in_specs=[
            pl.BlockSpec((1, 4, 2 * RP, PHW1, 128), lambda n, t: (n, 0, t, 0, 0)),
            pl.BlockSpec((1, 4, 1, PHW1, 128), lambda n, t: (n, 0, (t + 1) * 2 * RP, 0, 0)),
            pl.BlockSpec((9, 128, 128), lambda n, t: (0, 0, 0)),
            pl.BlockSpec((1, 128), lambda n, t: (0, 0)),
        ],
        out_specs=pl.BlockSpec((1, 2, RP, PHW1, 128), lambda n, t: (n, 0, t, 0, 0)),
        out_shape=jax.ShapeDtypeStruct((4, 2, 114, PHW1, 128), jnp.bfloat16),
    )(z1sq, z1sq, w2f, b2)


def _stageC_body(z_ref, w_ref, b_ref, cbt_ref, tok_ref, loss_ref):
    e2 = z_ref[0, 0].reshape(57, 2, PHW2, 128)
    o2 = z_ref[0, 1].reshape(57, 2, PHW2, 128)
    erow = [e2[:, 0], o2[:, 0]]
    orow = [e2[:, 1], o2[:, 1]]
    acc = jnp.zeros((56 * 56, 64), F32)
    for dy in range(3):
        for dx in range(3):
            if dy == 0:
                rs = [erow[0][0:56], erow[1][0:56]]
            elif dy == 1:
                rs = [orow[0][0:56], orow[1][0:56]]
            else:
                rs = [erow[0][1:57], erow[1][1:57]]
            if dx == 0:
                a = rs[0][:, 0:56]
            elif dx == 1:
                a = rs[1][:, 0:56]
            else:
                a = rs[0][:, 1:57]
            acc = acc + _matmul(a.reshape(56 * 56, 128), w_ref[dy * 3 + dx])
    z3 = acc + b_ref[0][None, :]
    cbt = cbt_ref[...]
    csq = jnp.sum(cbt * cbt, axis=0, keepdims=True)
    zsq = jnp.sum(z3 * z3, axis=1, keepdims=True)
    dots = _matmul(z3, cbt)
    dists = zsq + csq - 2.0 * dots
    tok_ref[0, 0] = jnp.argmin(dists, axis=1).astype(jnp.int32)
    loss_ref[0] = jnp.sum(jnp.min(dists, axis=1))[None, None]


def _stageC(z2s, w3f, b3, cbt):
    return pl.pallas_call(
        _stageC_body,
        grid=(4,),
        in_specs=[
            pl.BlockSpec((1, 2, 114, PHW2, 128), lambda n: (n, 0, 0, 0, 0)),
            pl.BlockSpec((9, 128, 64), lambda n: (0, 0, 0)),
            pl.BlockSpec((1, 64), lambda n: (0, 0)),
            pl.BlockSpec((64, 512), lambda n: (0, 0)),
        ],
        out_specs=[
            pl.BlockSpec((1, 1, 3136), lambda n: (n, 0, 0)),
            pl.BlockSpec((1, 1, 1), lambda n: (n, 0, 0)),
        ],
        out_shape=[
            jax.ShapeDtypeStruct((4, 1, 3136), jnp.int32),
            jax.ShapeDtypeStruct((4, 1, 1), F32),
        ],
    )(z2s, w3f, b3, cbt)


def kernel(force_field, w1, b1, w2, b2, w3, b3, codebook):
    x = jnp.transpose(force_field, (0, 2, 3, 1)).astype(jnp.bfloat16)
    xp = jnp.pad(x, ((0, 0), (4, 10), (2, 34), (0, 0)))
    xph = xp.reshape(4, 238, 65, 4, 6).transpose(0, 3, 1, 2, 4)
    phases = []
    for k in range(4):
        shifts = []
        for dy in range(3):
            for dx in range(3):
                pp, joff = (k + dx) % 4, (k + dx) // 4
                shifts.append(lax.slice(
                    xph, (0, pp, dy, joff, 0),
                    (4, pp + 1, dy + QROWS, joff + PHW1, 6))[:, 0])
        phases.append(jnp.concatenate(shifts, axis=3))
    xcat4 = jnp.stack(phases, axis=1).reshape(4, 4, NF1, 54)

    q = jnp.arange(NF1) // PHW1
    j4 = jnp.arange(NF1) % PHW1
    masks = []
    for k in range(4):
        c = 4 * j4 + k
        masks.append(((q >= 3) & (q < 227) & (c >= 1) & (c < 225)).astype(F32))
    mask = jnp.stack(masks, axis=0)[:, :, None]

    w1f = jnp.transpose(w1, (2, 3, 1, 0)).reshape(54, 128)
    w2f = jnp.transpose(w2, (2, 3, 1, 0)).reshape(9, 128, 128)
    w3f = jnp.transpose(w3, (2, 3, 1, 0)).reshape(9, 128, 64)
    b1r, b2r, b3r = b1[None, :], b2[None, :], b3[None, :]
    cbt = jnp.transpose(codebook, (1, 0))

    z1sq = _stageA(xcat4, w1f, b1r, mask)
    z1sq = z1sq.reshape(4, 4, QROWS, PHW1, 128)
    z2s = _stageB(z1sq, w2f, b2r)
    tok, loss_part = _stageC(z2s, w3f, b3r, cbt)

    tokens = tok.reshape(4, 56, 56)
    loss = (1.0 + 0.25) * jnp.sum(loss_part) / (4 * 64 * 56 * 56)
    return tokens, loss


# speedup vs baseline: 4.1833x; 1.2107x over previous
import jax
import jax.numpy as jnp
import numpy as np
from jax import lax
from jax.experimental import pallas as pl

F32 = jnp.float32

QROWS = 232
PHW1 = 64
NF1 = QROWS * PHW1
TILE_A = NF1 // 2
RP = 19
PHW2 = 64


def _matmul(a, w):
    return lax.dot_general(a.astype(jnp.bfloat16), w.astype(jnp.bfloat16),
                           (((a.ndim - 1,), (0,)), ((), ())),
                           preferred_element_type=F32)


def _silu(x):
    return x * jax.nn.sigmoid(x)


RQA = QROWS // 2


def _stageA_body(x_ref, w_ref, b_ref, o_ref):
    t = pl.program_id(1)
    bias = b_ref[0][None, :]
    i2 = lax.broadcasted_iota(jnp.int32, (TILE_A, 1), 0)
    q = t * RQA + i2 // PHW1
    j = i2 % PHW1
    rowok = (q >= 3) & (q < 227)
    for k in range(4):
        acc = jnp.zeros((TILE_A, 128), F32)
        for dy in range(3):
            for dx in range(3):
                pp, joff = (k + dx) % 4, (k + dx) // 4
                a = x_ref[0, pl.ds(t * RQA + dy, RQA),
                          pl.ds(joff, PHW1), pl.ds(6 * pp, 6)]
                acc = acc + _matmul(a.reshape(TILE_A, 6), w_ref[dy * 3 + dx])
        acc = _silu(acc + bias)
        colok = ((j >= 1) & (j < 57)) if k == 0 else (j < 56)
        ok = (rowok & colok).astype(F32)
        o_ref[0, k] = (acc * ok).astype(jnp.bfloat16)


def _stageA(xpc, w1f9, b1):
    return pl.pallas_call(
        _stageA_body,
        grid=(4, 2),
        in_specs=[
            pl.BlockSpec((1, 238, 65, 24), lambda n, t: (n, 0, 0, 0)),
            pl.BlockSpec((9, 6, 128), lambda n, t: (0, 0, 0)),
            pl.BlockSpec((1, 128), lambda n, t: (0, 0)),
        ],
        out_specs=pl.BlockSpec((1, 4, TILE_A, 128), lambda n, t: (n, 0, t, 0)),
        out_shape=jax.ShapeDtypeStruct((4, 4, NF1, 128), jnp.bfloat16),
    )(xpc, w1f9, b1)


def _stageB_body(m_ref, h_ref, w_ref, b_ref, o_ref):
    mr = m_ref[0].reshape(4, RP, 2, PHW1, 128)
    ev = mr[:, :, 0]
    od = mr[:, :, 1]
    halo = h_ref[0]
    slab = [ev, od, jnp.concatenate([ev[:, 1:RP], halo], axis=1)]
    bias = b_ref[0][None, :]
    t = pl.program_id(1)
    i2 = lax.broadcasted_iota(jnp.int32, (RP * PHW1, 1), 0)
    prow = t * RP + i2 // PHW1
    jcol = i2 % PHW1
    rowok = (prow >= 1) & (prow < 113)
    zero = jnp.zeros((RP, 1, 128), jnp.bfloat16)
    for par in range(2):
        acc = jnp.zeros((RP * PHW1, 128), F32)
        for dy in range(3):
            for dx in range(3):
                if par == 0:
                    if dx < 2:
                        a = jnp.concatenate([zero, slab[dy][2 + dx, :, 0:PHW1 - 1]], axis=1)
                    else:
                        a = slab[dy][0]
                else:
                    a = slab[dy][dx]
                acc = acc + _matmul(a.reshape(RP * PHW1, 128), w_ref[dy * 3 + dx])
        acc = _silu(acc + bias)
        if par == 0:
            ok = rowok & (jcol >= 1) & (jcol < 57)
        else:
            ok = rowok & (jcol < 56)
        acc = acc * ok.astype(F32)
        o_ref[0, par] = acc.reshape(RP, PHW1, 128).astype(jnp.bfloat16)


def _stageB(z1sq, w2f, b2):
    return pl.pallas_call(
        _stageB_body,
        grid=(4, 114 // RP),
        in_specs=[
            pl.BlockSpec((1, 4, 2 * RP, PHW1, 128), lambda n, t: (n, 0, t, 0, 0)),
            pl.BlockSpec((1, 4, 1, PHW1, 128), lambda n, t: (n, 0, (t + 1) * 2 * RP, 0, 0)),
            pl.BlockSpec((9, 128, 128), lambda n, t: (0, 0, 0)),
            pl.BlockSpec((1, 128), lambda n, t: (0, 0)),
        ],
        out_specs=pl.BlockSpec((1, 2, RP, PHW1, 128), lambda n, t: (n, 0, t, 0, 0)),
        out_shape=jax.ShapeDtypeStruct((4, 2, 114, PHW1, 128), jnp.bfloat16),
    )(z1sq, z1sq, w2f, b2)


def _stageC_body(z_ref, w_ref, b_ref, cbt_ref, tok_ref, loss_ref):
    e2 = z_ref[0, 0].reshape(57, 2, PHW2, 128)
    o2 = z_ref[0, 1].reshape(57, 2, PHW2, 128)
    erow = [e2[:, 0], o2[:, 0]]
    orow = [e2[:, 1], o2[:, 1]]
    acc = jnp.zeros((56 * 56, 64), F32)
    for dy in range(3):
        for dx in range(3):
            if dy == 0:
                rs = [erow[0][0:56], erow[1][0:56]]
            elif dy == 1:
                rs = [orow[0][0:56], orow[1][0:56]]
            else:
                rs = [erow[0][1:57], erow[1][1:57]]
            if dx == 0:
                a = rs[0][:, 0:56]
            elif dx == 1:
                a = rs[1][:, 0:56]
            else:
                a = rs[0][:, 1:57]
            acc = acc + _matmul(a.reshape(56 * 56, 128), w_ref[dy * 3 + dx])
    z3 = acc + b_ref[0][None, :]
    cbt = cbt_ref[...]
    csq = jnp.sum(cbt * cbt, axis=0, keepdims=True)
    zsq = jnp.sum(z3 * z3, axis=1, keepdims=True)
    dots = _matmul(z3, cbt)
    dists = zsq + csq - 2.0 * dots
    tok_ref[0, 0] = jnp.argmin(dists, axis=1).astype(jnp.int32)
    loss_ref[0] = jnp.sum(jnp.min(dists, axis=1))[None, None]


def _stageC(z2s, w3f, b3, cbt):
    return pl.pallas_call(
        _stageC_body,
        grid=(4,),
        in_specs=[
            pl.BlockSpec((1, 2, 114, PHW2, 128), lambda n: (n, 0, 0, 0, 0)),
            pl.BlockSpec((9, 128, 64), lambda n: (0, 0, 0)),
            pl.BlockSpec((1, 64), lambda n: (0, 0)),
            pl.BlockSpec((64, 512), lambda n: (0, 0)),
        ],
        out_specs=[
            pl.BlockSpec((1, 1, 3136), lambda n: (n, 0, 0)),
            pl.BlockSpec((1, 1, 1), lambda n: (n, 0, 0)),
        ],
        out_shape=[
            jax.ShapeDtypeStruct((4, 1, 3136), jnp.int32),
            jax.ShapeDtypeStruct((4, 1, 1), F32),
        ],
    )(z2s, w3f, b3, cbt)


def kernel(force_field, w1, b1, w2, b2, w3, b3, codebook):
    x = jnp.transpose(force_field, (0, 2, 3, 1)).astype(jnp.bfloat16)
    xp = jnp.pad(x, ((0, 0), (4, 10), (2, 34), (0, 0)))
    xpc = xp.reshape(4, 238, 65, 24)

    w1f9 = jnp.transpose(w1, (2, 3, 1, 0)).reshape(9, 6, 128)
    w2f = jnp.transpose(w2, (2, 3, 1, 0)).reshape(9, 128, 128)
    w3f = jnp.transpose(w3, (2, 3, 1, 0)).reshape(9, 128, 64)
    b1r, b2r, b3r = b1[None, :], b2[None, :], b3[None, :]
    cbt = jnp.transpose(codebook, (1, 0))

    z1sq = _stageA(xpc, w1f9, b1r)
    z1sq = z1sq.reshape(4, 4, QROWS, PHW1, 128)
    z2s = _stageB(z1sq, w2f, b2r)
    tok, loss_part = _stageC(z2s, w3f, b3r, cbt)

    tokens = tok.reshape(4, 56, 56)
    loss = (1.0 + 0.25) * jnp.sum(loss_part) / (4 * 64 * 56 * 56)
    return tokens, loss


# speedup vs baseline: 5.8035x; 1.3873x over previous
import jax
import jax.numpy as jnp
import numpy as np
from jax import lax
from jax.experimental import pallas as pl

F32 = jnp.float32

QROWS = 232
PHW1 = 64
NF1 = QROWS * PHW1
TILE_A = NF1 // 2
RP = 19
PHW2 = 64


def _matmul(a, w):
    return lax.dot_general(a.astype(jnp.bfloat16), w.astype(jnp.bfloat16),
                           (((a.ndim - 1,), (0,)), ((), ())),
                           preferred_element_type=F32)


def _silu(x):
    return x * jax.nn.sigmoid(x)


RQA = QROWS // 2


def _stageA_body(x_ref, w_ref, b_ref, o_ref):
    t = pl.program_id(1)
    bias = b_ref[0][None, :]
    i2 = lax.broadcasted_iota(jnp.int32, (TILE_A, 1), 0)
    q = t * RQA + i2 // PHW1
    j = i2 % PHW1
    rowok = (q >= 3) & (q < 227)
    a0 = [x_ref[0, pl.ds(t * RQA + dy, RQA), pl.ds(0, PHW1), :]
          .reshape(TILE_A, 32) for dy in range(3)]
    a1 = [x_ref[0, pl.ds(t * RQA + dy, RQA), pl.ds(1, PHW1), :]
          .reshape(TILE_A, 32) for dy in range(3)]
    for k in range(4):
        acc = jnp.zeros((TILE_A, 128), F32)
        for dy in range(3):
            acc = acc + _matmul(a0[dy], w_ref[k, dy, 0])
            if k >= 2:
                acc = acc + _matmul(a1[dy], w_ref[k, dy, 1])
        acc = _silu(acc + bias)
        colok = ((j >= 1) & (j < 57)) if k == 0 else (j < 56)
        ok = (rowok & colok).astype(F32)
        o_ref[0, k] = (acc * ok).astype(jnp.bfloat16)


def _stageA(xpc8, w1g, b1):
    return pl.pallas_call(
        _stageA_body,
        grid=(4, 2),
        in_specs=[
            pl.BlockSpec((1, 238, 65, 32), lambda n, t: (n, 0, 0, 0)),
            pl.BlockSpec((4, 3, 2, 32, 128), lambda n, t: (0, 0, 0, 0, 0)),
            pl.BlockSpec((1, 128), lambda n, t: (0, 0)),
        ],
        out_specs=pl.BlockSpec((1, 4, TILE_A, 128), lambda n, t: (n, 0, t, 0)),
        out_shape=jax.ShapeDtypeStruct((4, 4, NF1, 128), jnp.bfloat16),
    )(xpc8, w1g, b1)


def _stageB_body(m_ref, h_ref, w_ref, b_ref, o_ref):
    mr = m_ref[0].reshape(4, RP, 2, PHW1, 128)
    ev = mr[:, :, 0]
    od = mr[:, :, 1]
    halo = h_ref[0]
    slab = [ev, od, jnp.concatenate([ev[:, 1:RP], halo], axis=1)]
    bias = b_ref[0][None, :]
    t = pl.program_id(1)
    i2 = lax.broadcasted_iota(jnp.int32, (RP * PHW1, 1), 0)
    prow = t * RP + i2 // PHW1
    jcol = i2 % PHW1
    rowok = (prow >= 1) & (prow < 113)
    zero = jnp.zeros((RP, 1, 128), jnp.bfloat16)
    for par in range(2):
        acc = jnp.zeros((RP * PHW1, 128), F32)
        for dy in range(3):
            for dx in range(3):
                if par == 0:
                    if dx < 2:
                        a = jnp.concatenate([zero, slab[dy][2 + dx, :, 0:PHW1 - 1]], axis=1)
                    else:
                        a = slab[dy][0]
                else:
                    a = slab[dy][dx]
                acc = acc + _matmul(a.reshape(RP * PHW1, 128), w_ref[dy * 3 + dx])
        acc = _silu(acc + bias)
        if par == 0:
            ok = rowok & (jcol >= 1) & (jcol < 57)
        else:
            ok = rowok & (jcol < 56)
        acc = acc * ok.astype(F32)
        o_ref[0, par] = acc.reshape(RP, PHW1, 128).astype(jnp.bfloat16)


def _stageB(z1sq, w2f, b2):
    return pl.pallas_call(
        _stageB_body,
        grid=(4, 114 // RP),
        in_specs=[
            pl.BlockSpec((1, 4, 2 * RP, PHW1, 128), lambda n, t: (n, 0, t, 0, 0)),
            pl.BlockSpec((1, 4, 1, PHW1, 128), lambda n, t: (n, 0, (t + 1) * 2 * RP, 0, 0)),
            pl.BlockSpec((9, 128, 128), lambda n, t: (0, 0, 0)),
            pl.BlockSpec((1, 128), lambda n, t: (0, 0)),
        ],
        out_specs=pl.BlockSpec((1, 2, RP, PHW1, 128), lambda n, t: (n, 0, t, 0, 0)),
        out_shape=jax.ShapeDtypeStruct((4, 2, 114, PHW1, 128), jnp.bfloat16),
    )(z1sq, z1sq, w2f, b2)


def _stageC_body(z_ref, w_ref, b_ref, cbt_ref, tok_ref, loss_ref):
    e2 = z_ref[0, 0].reshape(57, 2, PHW2, 128)
    o2 = z_ref[0, 1].reshape(57, 2, PHW2, 128)
    erow = [e2[:, 0], o2[:, 0]]
    orow = [e2[:, 1], o2[:, 1]]
    acc = jnp.zeros((56 * 56, 64), F32)
    for dy in range(3):
        for dx in range(3):
            if dy == 0:
                rs = [erow[0][0:56], erow[1][0:56]]
            elif dy == 1:
                rs = [orow[0][0:56], orow[1][0:56]]
            else:
                rs = [erow[0][1:57], erow[1][1:57]]
            if dx == 0:
                a = rs[0][:, 0:56]
            elif dx == 1:
                a = rs[1][:, 0:56]
            else:
                a = rs[0][:, 1:57]
            acc = acc + _matmul(a.reshape(56 * 56, 128), w_ref[dy * 3 + dx])
    z3 = acc + b_ref[0][None, :]
    cbt = cbt_ref[...]
    csq = jnp.sum(cbt * cbt, axis=0, keepdims=True)
    zsq = jnp.sum(z3 * z3, axis=1, keepdims=True)
    dots = _matmul(z3, cbt)
    dists = zsq + csq - 2.0 * dots
    tok_ref[0, 0] = jnp.argmin(dists, axis=1).astype(jnp.int32)
    loss_ref[0] = jnp.sum(jnp.min(dists, axis=1))[None, None]


def _stageC(z2s, w3f, b3, cbt):
    return pl.pallas_call(
        _stageC_body,
        grid=(4,),
        in_specs=[
            pl.BlockSpec((1, 2, 114, PHW2, 128), lambda n: (n, 0, 0, 0, 0)),
            pl.BlockSpec((9, 128, 64), lambda n: (0, 0, 0)),
            pl.BlockSpec((1, 64), lambda n: (0, 0)),
            pl.BlockSpec((64, 512), lambda n: (0, 0)),
        ],
        out_specs=[
            pl.BlockSpec((1, 1, 3136), lambda n: (n, 0, 0)),
            pl.BlockSpec((1, 1, 1), lambda n: (n, 0, 0)),
        ],
        out_shape=[
            jax.ShapeDtypeStruct((4, 1, 3136), jnp.int32),
            jax.ShapeDtypeStruct((4, 1, 1), F32),
        ],
    )(z2s, w3f, b3, cbt)


def kernel(force_field, w1, b1, w2, b2, w3, b3, codebook):
    x = jnp.transpose(force_field, (0, 2, 3, 1)).astype(jnp.bfloat16)
    xp = jnp.pad(x, ((0, 0), (4, 10), (2, 34), (0, 2)))
    xpc8 = xp.reshape(4, 238, 65, 32)

    w1t = jnp.transpose(w1, (2, 3, 1, 0))
    w1g = jnp.zeros((4, 3, 2, 32, 128), F32)
    for k in range(4):
        for p in range(4):
            for joff in range(2):
                dx = p + 4 * joff - k
                if 0 <= dx < 3:
                    w1g = w1g.at[k, :, joff, 8 * p:8 * p + 6, :].set(w1t[:, dx])
    w2f = jnp.transpose(w2, (2, 3, 1, 0)).reshape(9, 128, 128)
    w3f = jnp.transpose(w3, (2, 3, 1, 0)).reshape(9, 128, 64)
    b1r, b2r, b3r = b1[None, :], b2[None, :], b3[None, :]
    cbt = jnp.transpose(codebook, (1, 0))

    z1sq = _stageA(xpc8, w1g, b1r)
    z1sq = z1sq.reshape(4, 4, QROWS, PHW1, 128)
    z2s = _stageB(z1sq, w2f, b2r)
    tok, loss_part = _stageC(z2s, w3f, b3r, cbt)

    tokens = tok.reshape(4, 56, 56)
    loss = (1.0 + 0.25) * jnp.sum(loss_part) / (4 * 64 * 56 * 56)
    return tokens, loss


# speedup vs baseline: 5.9232x; 1.0206x over previous
import jax
import jax.numpy as jnp
import numpy as np
from jax import lax
from jax.experimental import pallas as pl

F32 = jnp.float32

QROWS = 232
PHW1 = 64
NF1 = QROWS * PHW1
TILE_A = NF1 // 2
RP = 19
PHW2 = 64


def _matmul(a, w):
    return lax.dot_general(a.astype(jnp.bfloat16), w.astype(jnp.bfloat16),
                           (((a.ndim - 1,), (0,)), ((), ())),
                           preferred_element_type=F32)


def _silu(x):
    return x * jax.nn.sigmoid(x)


RQA = QROWS // 2


def _stageA_body(x_ref, w_ref, b_ref, o_ref):
    t = pl.program_id(1)
    bias = b_ref[0][None, :]
    i2 = lax.broadcasted_iota(jnp.int32, (TILE_A, 1), 0)
    q = t * RQA + i2 // PHW1
    j = i2 % PHW1
    rowok = (q >= 3) & (q < 227)
    a0 = [x_ref[0, pl.ds(t * RQA + dy, RQA), pl.ds(0, PHW1), :]
          .reshape(TILE_A, 32) for dy in range(3)]
    a1 = [x_ref[0, pl.ds(t * RQA + dy, RQA), pl.ds(1, PHW1), :]
          .reshape(TILE_A, 32) for dy in range(3)]
    for k in range(4):
        acc = jnp.zeros((TILE_A, 128), F32)
        for dy in range(3):
            acc = acc + _matmul(a0[dy], w_ref[k, dy, 0])
            if k >= 2:
                acc = acc + _matmul(a1[dy], w_ref[k, dy, 1])
        acc = _silu(acc + bias)
        colok = ((j >= 1) & (j < 57)) if k == 0 else (j < 56)
        ok = (rowok & colok).astype(F32)
        o_ref[0, k] = (acc * ok).astype(jnp.bfloat16)


def _stageA(xpc8, w1g, b1):
    return pl.pallas_call(
        _stageA_body,
        grid=(4, 2),
        in_specs=[
            pl.BlockSpec((1, 238, 65, 32), lambda n, t: (n, 0, 0, 0)),
            pl.BlockSpec((4, 3, 2, 32, 128), lambda n, t: (0, 0, 0, 0, 0)),
            pl.BlockSpec((1, 128), lambda n, t: (0, 0)),
        ],
        out_specs=pl.BlockSpec((1, 4, TILE_A, 128), lambda n, t: (n, 0, t, 0)),
        out_shape=jax.ShapeDtypeStruct((4, 4, NF1, 128), jnp.bfloat16),
    )(xpc8, w1g, b1)


def _stageB_body(m_ref, h_ref, w_ref, b_ref, o_ref):
    mr = m_ref[0].reshape(4, RP, 2, PHW1, 128)
    ev = mr[:, :, 0]
    od = mr[:, :, 1]
    halo = h_ref[0]
    slab = [ev, od, jnp.concatenate([ev[:, 1:RP], halo], axis=1)]
    bias = b_ref[0][None, :]
    t = pl.program_id(1)
    i2 = lax.broadcasted_iota(jnp.int32, (RP * PHW1, 1), 0)
    prow = t * RP + i2 // PHW1
    jcol = i2 % PHW1
    rowok = (prow >= 1) & (prow < 113)
    zero = jnp.zeros((RP, 1, 128), jnp.bfloat16)
    for par in range(2):
        ops = []
        for dy in range(3):
            for dx in range(3):
                if par == 0:
                    if dx < 2:
                        a = jnp.concatenate([zero, slab[dy][2 + dx, :, 0:PHW1 - 1]], axis=1)
                    else:
                        a = slab[dy][0]
                else:
                    a = slab[dy][dx]
                ops.append(a.reshape(RP * PHW1, 128))
        acc = _matmul(jnp.concatenate(ops, axis=1), w_ref[...])
        acc = _silu(acc + bias)
        if par == 0:
            ok = rowok & (jcol >= 1) & (jcol < 57)
        else:
            ok = rowok & (jcol < 56)
        acc = acc * ok.astype(F32)
        o_ref[0, par] = acc.reshape(RP, PHW1, 128).astype(jnp.bfloat16)


def _stageB(z1sq, w2f, b2):
    return pl.pallas_call(
        _stageB_body,
        grid=(4, 114 // RP),
        in_specs=[
            pl.BlockSpec((1, 4, 2 * RP, PHW1, 128), lambda n, t: (n, 0, t, 0, 0)),
            pl.BlockSpec((1, 4, 1, PHW1, 128), lambda n, t: (n, 0, (t + 1) * 2 * RP, 0, 0)),
            pl.BlockSpec((1152, 128), lambda n, t: (0, 0)),
            pl.BlockSpec((1, 128), lambda n, t: (0, 0)),
        ],
        out_specs=pl.BlockSpec((1, 2, RP, PHW1, 128), lambda n, t: (n, 0, t, 0, 0)),
        out_shape=jax.ShapeDtypeStruct((4, 2, 114, PHW1, 128), jnp.bfloat16),
    )(z1sq, z1sq, w2f, b2)


def _stageC_body(z_ref, w_ref, b_ref, cbt_ref, tok_ref, loss_ref):
    e2 = z_ref[0, 0].reshape(57, 2, PHW2, 128)
    o2 = z_ref[0, 1].reshape(57, 2, PHW2, 128)
    erow = [e2[:, 0], o2[:, 0]]
    orow = [e2[:, 1], o2[:, 1]]
    ops = []
    for dy in range(3):
        for dx in range(3):
            if dy == 0:
                rs = [erow[0][0:56], erow[1][0:56]]
            elif dy == 1:
                rs = [orow[0][0:56], orow[1][0:56]]
            else:
                rs = [erow[0][1:57], erow[1][1:57]]
            if dx == 0:
                a = rs[0][:, 0:56]
            elif dx == 1:
                a = rs[1][:, 0:56]
            else:
                a = rs[0][:, 1:57]
            ops.append(a.reshape(56 * 56, 128))
    acc = _matmul(jnp.concatenate(ops, axis=1), w_ref[...])
    z3 = acc + b_ref[0][None, :]
    cbt = cbt_ref[...]
    csq = jnp.sum(cbt * cbt, axis=0, keepdims=True)
    zsq = jnp.sum(z3 * z3, axis=1, keepdims=True)
    dots = _matmul(z3, cbt)
    dists = zsq + csq - 2.0 * dots
    tok_ref[0, 0] = jnp.argmin(dists, axis=1).astype(jnp.int32)
    loss_ref[0] = jnp.sum(jnp.min(dists, axis=1))[None, None]


def _stageC(z2s, w3f, b3, cbt):
    return pl.pallas_call(
        _stageC_body,
        grid=(4,),
        in_specs=[
            pl.BlockSpec((1, 2, 114, PHW2, 128), lambda n: (n, 0, 0, 0, 0)),
            pl.BlockSpec((1152, 64), lambda n: (0, 0)),
            pl.BlockSpec((1, 64), lambda n: (0, 0)),
            pl.BlockSpec((64, 512), lambda n: (0, 0)),
        ],
        out_specs=[
            pl.BlockSpec((1, 1, 3136), lambda n: (n, 0, 0)),
            pl.BlockSpec((1, 1, 1), lambda n: (n, 0, 0)),
        ],
        out_shape=[
            jax.ShapeDtypeStruct((4, 1, 3136), jnp.int32),
            jax.ShapeDtypeStruct((4, 1, 1), F32),
        ],
    )(z2s, w3f, b3, cbt)


def kernel(force_field, w1, b1, w2, b2, w3, b3, codebook):
    x = jnp.transpose(force_field, (0, 2, 3, 1)).astype(jnp.bfloat16)
    xp = jnp.pad(x, ((0, 0), (4, 10), (2, 34), (0, 2)))
    xpc8 = xp.reshape(4, 238, 65, 32)

    w1t = jnp.transpose(w1, (2, 3, 1, 0))
    w1g = jnp.zeros((4, 3, 2, 32, 128), F32)
    for k in range(4):
        for p in range(4):
            for joff in range(2):
                dx = p + 4 * joff - k
                if 0 <= dx < 3:
                    w1g = w1g.at[k, :, joff, 8 * p:8 * p + 6, :].set(w1t[:, dx])
    w2f = jnp.transpose(w2, (2, 3, 1, 0)).reshape(1152, 128)
    w3f = jnp.transpose(w3, (2, 3, 1, 0)).reshape(1152, 64)
    b1r, b2r, b3r = b1[None, :], b2[None, :], b3[None, :]
    cbt = jnp.transpose(codebook, (1, 0))

    z1sq = _stageA(xpc8, w1g, b1r)
    z1sq = z1sq.reshape(4, 4, QROWS, PHW1, 128)
    z2s = _stageB(z1sq, w2f, b2r)
    tok, loss_part = _stageC(z2s, w3f, b3r, cbt)

    tokens = tok.reshape(4, 56, 56)
    loss = (1.0 + 0.25) * jnp.sum(loss_part) / (4 * 64 * 56 * 56)
    return tokens, loss
